# Initial kernel scaffold; baseline (speedup 1.0000x reference)
import functools

import jax
import jax.numpy as jnp
from jax import lax
from jax.experimental import pallas as pl
from jax.experimental.pallas import tpu as pltpu
from jax.experimental.pallas import tpu_sc as plsc

V = 1000
V_PAD = 1024
BT = 20480
NC, NS, L = 2, 16, 16
NW = NC * NS
B_PER_W = BT // NW
CHUNK = 64
N_CHUNKS = B_PER_W // CHUNK


def _lse_body(table_ref, lse_ref):
    x = table_ref[...]
    m = jnp.max(x, axis=1)
    s = jnp.sum(jnp.exp(x - m[:, None]), axis=1)
    lse_ref[pl.ds(0, V)] = m + jnp.log(s)
    lse_ref[pl.ds(V, V_PAD - V)] = jnp.zeros((V_PAD - V,), jnp.float32)


def _row_lse(table):
    return pl.pallas_call(
        _lse_body,
        out_shape=jax.ShapeDtypeStruct((V_PAD,), jnp.float32),
    )(table)


_MESH = plsc.VectorSubcoreMesh(core_axis_name="c", subcore_axis_name="s")


@functools.partial(
    pl.kernel,
    mesh=_MESH,
    out_type=[
        jax.ShapeDtypeStruct((BT, V), jnp.float32),
        jax.ShapeDtypeStruct((NW, L), jnp.float32),
    ],
    scratch_types=[
        pltpu.VMEM((B_PER_W,), jnp.int32),
        pltpu.VMEM((B_PER_W,), jnp.int32),
        pltpu.VMEM((V_PAD,), jnp.float32),
        pltpu.VMEM((CHUNK, V), jnp.float32),
        pltpu.VMEM((L,), jnp.float32),
        pltpu.SemaphoreType.DMA,
    ],
)
def _sc_gather(table_hbm, idx_hbm, tgt_hbm, lse_hbm, out_hbm, loss_hbm,
               idx_v, tgt_v, lse_v, rows_v, acc_v, sem):
    wid = lax.axis_index("s") * NC + lax.axis_index("c")
    base = wid * B_PER_W
    pltpu.sync_copy(idx_hbm.at[pl.ds(base, B_PER_W)], idx_v)
    pltpu.sync_copy(tgt_hbm.at[pl.ds(base, B_PER_W)], tgt_v)
    pltpu.sync_copy(lse_hbm, lse_v)
    lane = lax.iota(jnp.int32, L)
    acc = jnp.zeros((L,), jnp.float32)
    for c in range(N_CHUNKS):
        pltpu.async_copy(
            table_hbm.at[idx_v.at[pl.ds(c * CHUNK, CHUNK)]], rows_v, sem
        ).wait()
        pltpu.sync_copy(rows_v, out_hbm.at[pl.ds(base + c * CHUNK, CHUNK)])
        for j in range(CHUNK // L):
            p = c * CHUNK + j * L
            ids = idx_v[pl.ds(p, L)]
            tgs = tgt_v[pl.ds(p, L)]
            vals = plsc.load_gather(rows_v, [lane + (j * L), tgs])
            lses = plsc.load_gather(lse_v, [ids])
            acc = acc + (lses - vals)
    acc_v[...] = acc * (1.0 / BT)
    pltpu.sync_copy(acc_v, loss_hbm.at[wid])


def kernel(idx, targets, table):
    idx_f = idx.reshape(-1).astype(jnp.int32)
    tgt_f = targets.reshape(-1).astype(jnp.int32)
    lse = _row_lse(table)
    logits2, loss_part = _sc_gather(table, idx_f, tgt_f, lse)
    loss = jnp.sum(loss_part)
    return (logits2, loss)


# speedup vs baseline: 1.7181x; 1.7181x over previous
import functools

import jax
import jax.numpy as jnp
from jax import lax
from jax.experimental import pallas as pl
from jax.experimental.pallas import tpu as pltpu
from jax.experimental.pallas import tpu_sc as plsc

V = 1000
V_PAD = 1024
BT = 20480
NC, NS, L = 2, 16, 16
NW = NC * NS
B_PER_W = BT // NW
CHUNK = 64
N_CHUNKS = B_PER_W // CHUNK


def _lse_body(table_ref, lse_ref):
    x = table_ref[...]
    m = jnp.max(x, axis=1)
    s = jnp.sum(jnp.exp(x - m[:, None]), axis=1)
    lse_ref[pl.ds(0, V)] = m + jnp.log(s)
    lse_ref[pl.ds(V, V_PAD - V)] = jnp.zeros((V_PAD - V,), jnp.float32)


def _row_lse(table):
    return pl.pallas_call(
        _lse_body,
        out_shape=jax.ShapeDtypeStruct((V_PAD,), jnp.float32),
    )(table)


_MESH = plsc.VectorSubcoreMesh(core_axis_name="c", subcore_axis_name="s")


@functools.partial(
    pl.kernel,
    mesh=_MESH,
    compiler_params=pltpu.CompilerParams(
        use_tc_tiling_on_sc=False, needs_layout_passes=False
    ),
    out_type=[
        jax.ShapeDtypeStruct((BT, V), jnp.float32),
        jax.ShapeDtypeStruct((NW, L), jnp.float32),
    ],
    scratch_types=[
        pltpu.VMEM((B_PER_W,), jnp.int32),
        pltpu.VMEM((B_PER_W,), jnp.int32),
        pltpu.VMEM((V_PAD,), jnp.float32),
        pltpu.VMEM((CHUNK, V), jnp.float32),
        pltpu.VMEM((L,), jnp.float32),
        pltpu.SemaphoreType.DMA,
    ],
)
def _sc_gather(table_hbm, idx_hbm, tgt_hbm, lse_hbm, out_hbm, loss_hbm,
               idx_v, tgt_v, lse_v, rows_v, acc_v, sem):
    wid = lax.axis_index("s") * NC + lax.axis_index("c")
    base = wid * B_PER_W
    pltpu.sync_copy(idx_hbm.at[pl.ds(base, B_PER_W)], idx_v)
    pltpu.sync_copy(tgt_hbm.at[pl.ds(base, B_PER_W)], tgt_v)
    pltpu.sync_copy(lse_hbm, lse_v)
    lane = lax.iota(jnp.int32, L)
    acc = jnp.zeros((L,), jnp.float32)
    for c in range(N_CHUNKS):
        pltpu.async_copy(
            table_hbm.at[idx_v.at[pl.ds(c * CHUNK, CHUNK)]], rows_v, sem
        ).wait()
        pltpu.sync_copy(rows_v, out_hbm.at[pl.ds(base + c * CHUNK, CHUNK)])
        for j in range(CHUNK // L):
            p = c * CHUNK + j * L
            ids = idx_v[pl.ds(p, L)]
            tgs = tgt_v[pl.ds(p, L)]
            vals = plsc.load_gather(rows_v, [lane + (j * L), tgs])
            lses = plsc.load_gather(lse_v, [ids])
            acc = acc + (lses - vals)
    acc_v[...] = acc * (1.0 / BT)
    pltpu.sync_copy(acc_v, loss_hbm.at[wid])


def kernel(idx, targets, table):
    idx_f = idx.reshape(-1).astype(jnp.int32)
    tgt_f = targets.reshape(-1).astype(jnp.int32)
    lse = _row_lse(table)
    logits2, loss_part = _sc_gather(table, idx_f, tgt_f, lse)
    loss = jnp.sum(loss_part)
    return (logits2, loss)


# speedup vs baseline: 1.7435x; 1.0148x over previous
import functools

import jax
import jax.numpy as jnp
from jax import lax
from jax.experimental import pallas as pl
from jax.experimental.pallas import tpu as pltpu
from jax.experimental.pallas import tpu_sc as plsc

V = 1000
V_PAD = 1024
BT = 20480
NC, NS, L = 2, 16, 16
NW = NC * NS
B_PER_W = BT // NW
CHUNK = 64
N_CHUNKS = B_PER_W // CHUNK


def _lse_body(table_ref, lse_ref):
    x = table_ref[...]
    m = jnp.max(x, axis=1)
    s = jnp.sum(jnp.exp(x - m[:, None]), axis=1)
    lse_ref[pl.ds(0, V)] = m + jnp.log(s)
    lse_ref[pl.ds(V, V_PAD - V)] = jnp.zeros((V_PAD - V,), jnp.float32)


def _row_lse(table):
    return pl.pallas_call(
        _lse_body,
        out_shape=jax.ShapeDtypeStruct((V_PAD,), jnp.float32),
    )(table)


_MESH = plsc.VectorSubcoreMesh(core_axis_name="c", subcore_axis_name="s")


@functools.partial(
    pl.kernel,
    mesh=_MESH,
    compiler_params=pltpu.CompilerParams(
        use_tc_tiling_on_sc=False, needs_layout_passes=False
    ),
    out_type=[
        jax.ShapeDtypeStruct((BT, V), jnp.float32),
        jax.ShapeDtypeStruct((NW, L), jnp.float32),
    ],
    scratch_types=[
        pltpu.VMEM((B_PER_W,), jnp.int32),
        pltpu.VMEM((B_PER_W,), jnp.int32),
        pltpu.VMEM((V_PAD,), jnp.float32),
        pltpu.VMEM((CHUNK, V), jnp.float32),
        pltpu.VMEM((CHUNK, V), jnp.float32),
        pltpu.VMEM((L,), jnp.float32),
        pltpu.SemaphoreType.DMA,
        pltpu.SemaphoreType.DMA,
    ],
)
def _sc_gather(table_hbm, idx_hbm, tgt_hbm, lse_hbm, out_hbm, loss_hbm,
               idx_v, tgt_v, lse_v, rows0_v, rows1_v, acc_v, sem0, sem1):
    wid = lax.axis_index("s") * NC + lax.axis_index("c")
    base = wid * B_PER_W
    pltpu.sync_copy(idx_hbm.at[pl.ds(base, B_PER_W)], idx_v)
    pltpu.sync_copy(tgt_hbm.at[pl.ds(base, B_PER_W)], tgt_v)
    pltpu.sync_copy(lse_hbm, lse_v)
    lane = lax.iota(jnp.int32, L)
    acc = jnp.zeros((L,), jnp.float32)
    bufs = (rows0_v, rows1_v)
    sems = (sem0, sem1)
    pending = pltpu.async_copy(
        table_hbm.at[idx_v.at[pl.ds(0, CHUNK)]], rows0_v, sem0
    )
    for c in range(N_CHUNKS):
        buf = bufs[c % 2]
        pending.wait()
        if c + 1 < N_CHUNKS:
            pending = pltpu.async_copy(
                table_hbm.at[idx_v.at[pl.ds((c + 1) * CHUNK, CHUNK)]],
                bufs[(c + 1) % 2], sems[(c + 1) % 2],
            )
        pltpu.sync_copy(buf, out_hbm.at[pl.ds(base + c * CHUNK, CHUNK)])
        for j in range(CHUNK // L):
            p = c * CHUNK + j * L
            ids = idx_v[pl.ds(p, L)]
            tgs = tgt_v[pl.ds(p, L)]
            vals = plsc.load_gather(buf, [lane + (j * L), tgs])
            lses = plsc.load_gather(lse_v, [ids])
            acc = acc + (lses - vals)
    acc_v[...] = acc * (1.0 / BT)
    pltpu.sync_copy(acc_v, loss_hbm.at[wid])


def kernel(idx, targets, table):
    idx_f = idx.reshape(-1).astype(jnp.int32)
    tgt_f = targets.reshape(-1).astype(jnp.int32)
    lse = _row_lse(table)
    logits2, loss_part = _sc_gather(table, idx_f, tgt_f, lse)
    loss = jnp.sum(loss_part)
    return (logits2, loss)
